# Initial kernel scaffold; baseline (speedup 1.0000x reference)
import jax
import jax.numpy as jnp
from jax.experimental import pallas as pl


def kernel(x, edge_index, batch, params):
    raise NotImplementedError("write your pallas kernel here")


# speedup vs baseline: 7.8214x; 7.8214x over previous
import functools

import jax
import jax.numpy as jnp
from jax import lax
from jax.experimental import pallas as pl
from jax.experimental.pallas import tpu as pltpu
from jax.experimental.pallas import tpu_sc as plsc

N = 10000
NPAD = 10240
E = 320000
D = 128
HID = 128
FFN = 256
G = 64
NW = 32
EPW = E // NW
CH = 128
NFULL = EPW // CH
TAIL = EPW - NFULL * CH
RPT = NPAD // 16

DRB = 1024
PRB = 1024
RBR = 512
CBR = 512

def _mesh():
    return plsc.VectorSubcoreMesh(core_axis_name="c", subcore_axis_name="s")


def _ln_in(x, g, b):
    m = jnp.mean(x, axis=-1, keepdims=True)
    d = x - m
    v = jnp.mean(d * d, axis=-1, keepdims=True)
    return d / jnp.sqrt(v + 1e-5) * g + b


def _embed_body(x_ref, w_ref, b_ref, o_ref):
    o_ref[...] = (
        jnp.dot(x_ref[...], w_ref[...], preferred_element_type=jnp.float32)
        + b_ref[...]
    )


def _embed(x, w, b):
    return pl.pallas_call(
        _embed_body,
        grid=(NPAD // DRB,),
        in_specs=[
            pl.BlockSpec((DRB, D), lambda i: (i, 0)),
            pl.BlockSpec((D, HID), lambda i: (0, 0)),
            pl.BlockSpec((1, HID), lambda i: (0, 0)),
        ],
        out_specs=pl.BlockSpec((DRB, HID), lambda i: (i, 0)),
        out_shape=jax.ShapeDtypeStruct((NPAD, HID), jnp.float32),
    )(x, w, b.reshape(1, HID))


def _dense1_body(x_ref, lng, lnb, w1, b1, sgg, sgb, u_ref, v_ref):
    h = _ln_in(x_ref[...], lng[...], lnb[...])
    h = jax.nn.gelu(
        jnp.dot(h, w1[...], preferred_element_type=jnp.float32) + b1[...]
    )
    u_ref[...] = h[:, :HID]
    v_ref[...] = _ln_in(h[:, HID:], sgg[...], sgb[...])


def _dense1(x, p):
    vec = lambda a, n: a.reshape(1, n)
    return pl.pallas_call(
        _dense1_body,
        grid=(NPAD // DRB,),
        in_specs=[
            pl.BlockSpec((DRB, HID), lambda i: (i, 0)),
            pl.BlockSpec((1, HID), lambda i: (0, 0)),
            pl.BlockSpec((1, HID), lambda i: (0, 0)),
            pl.BlockSpec((HID, FFN), lambda i: (0, 0)),
            pl.BlockSpec((1, FFN), lambda i: (0, 0)),
            pl.BlockSpec((1, FFN // 2), lambda i: (0, 0)),
            pl.BlockSpec((1, FFN // 2), lambda i: (0, 0)),
        ],
        out_specs=[
            pl.BlockSpec((DRB, HID), lambda i: (i, 0)),
            pl.BlockSpec((DRB, HID), lambda i: (i, 0)),
        ],
        out_shape=[
            jax.ShapeDtypeStruct((NPAD, HID), jnp.float32),
            jax.ShapeDtypeStruct((NPAD, HID), jnp.float32),
        ],
    )(x, vec(p["ln_g"], HID), vec(p["ln_b"], HID), p["W1"], vec(p["b1"], FFN),
      vec(p["sg_g"], FFN // 2), vec(p["sg_b"], FFN // 2))


def _dense2_body(with_score, u_ref, a_ref, dg_ref, sb, w2, b2, sc_ref, pw_ref,
                 x_ref, *maybe_xrn):
    deg = jnp.maximum(dg_ref[0] + dg_ref[1], 1.0)
    v = (a_ref[0] + a_ref[1]) / deg + sb[...]
    h = (
        jnp.dot(u_ref[...] * v, w2[...], preferred_element_type=jnp.float32)
        + b2[...]
        + sc_ref[...]
    )
    x_ref[...] = h
    if with_score:
        maybe_xrn[0][...] = jnp.dot(
            h, pw_ref[...], preferred_element_type=jnp.float32
        )


def _dense2(u, agg2, deg2c, shortcut, p, pw):
    with_score = pw is not None
    if pw is None:
        pw = jnp.zeros((HID, 8), jnp.float32)
    vec = lambda a, n: a.reshape(1, n)
    out_shape = [jax.ShapeDtypeStruct((NPAD, HID), jnp.float32)]
    out_specs = [pl.BlockSpec((DRB, HID), lambda i: (i, 0))]
    if with_score:
        out_shape.append(jax.ShapeDtypeStruct((NPAD, 8), jnp.float32))
        out_specs.append(pl.BlockSpec((DRB, 8), lambda i: (i, 0)))
    outs = pl.pallas_call(
        functools.partial(_dense2_body, with_score),
        grid=(NPAD // DRB,),
        in_specs=[
            pl.BlockSpec((DRB, HID), lambda i: (i, 0)),
            pl.BlockSpec((2, DRB, HID), lambda i: (0, i, 0)),
            pl.BlockSpec((2, DRB, 1), lambda i: (0, i, 0)),
            pl.BlockSpec((1, FFN // 2), lambda i: (0, 0)),
            pl.BlockSpec((FFN // 2, HID), lambda i: (0, 0)),
            pl.BlockSpec((1, HID), lambda i: (0, 0)),
            pl.BlockSpec((DRB, HID), lambda i: (i, 0)),
            pl.BlockSpec((HID, 8), lambda i: (0, 0)),
        ],
        out_specs=out_specs,
        out_shape=out_shape,
    )(u, agg2, deg2c, vec(p["sb"], FFN // 2), p["W2"], vec(p["b2"], HID),
      shortcut, pw)
    return outs if with_score else (outs[0], None)


def _pool_body(bounds_ref, batr_ref, valr_ref, batc_ref, valc_ref, y_ref,
               sum_ref, max_ref, cnt_ref):
    i = pl.program_id(0)

    @pl.when(i == 0)
    def _():
        sum_ref[...] = jnp.zeros_like(sum_ref)
        max_ref[...] = jnp.full_like(max_ref, -jnp.inf)
        cnt_ref[...] = jnp.zeros_like(cnt_ref)

    y = y_ref[...]
    oh = (
        (lax.broadcasted_iota(jnp.int32, (G, 1), 0) == batr_ref[...])
        & (valr_ref[...] > 0.0)
    ).astype(jnp.float32)
    sum_ref[...] += jnp.dot(oh, y, preferred_element_type=jnp.float32)
    cnt_ref[...] += jnp.sum(oh, axis=1, keepdims=True)

    batc = batc_ref[...]
    valc = valc_ref[...]

    def body(g, _):
        m = (batc == g) & (valc > 0.0)
        v = jnp.max(jnp.where(m, y, -jnp.inf), axis=0, keepdims=True)
        max_ref[pl.ds(g, 1), :] = jnp.maximum(max_ref[pl.ds(g, 1), :], v)
        return 0

    lax.fori_loop(bounds_ref[i, 0], bounds_ref[i, 1] + 1, body, 0)


def _pool(y, batr, valr, batc, valc, bounds):
    return pl.pallas_call(
        _pool_body,
        grid=(NPAD // PRB,),
        in_specs=[
            pl.BlockSpec(memory_space=pltpu.SMEM),
            pl.BlockSpec((1, PRB), lambda i: (0, i)),
            pl.BlockSpec((1, PRB), lambda i: (0, i)),
            pl.BlockSpec((PRB, 1), lambda i: (i, 0)),
            pl.BlockSpec((PRB, 1), lambda i: (i, 0)),
            pl.BlockSpec((PRB, HID), lambda i: (i, 0)),
        ],
        out_specs=[
            pl.BlockSpec((G, HID), lambda i: (0, 0)),
            pl.BlockSpec((G, HID), lambda i: (0, 0)),
            pl.BlockSpec((G, 1), lambda i: (0, 0)),
        ],
        out_shape=[
            jax.ShapeDtypeStruct((G, HID), jnp.float32),
            jax.ShapeDtypeStruct((G, HID), jnp.float32),
            jax.ShapeDtypeStruct((G, 1), jnp.float32),
        ],
    )(bounds, batr, valr, batc, valc, y)


def _rank_body(cb_ref, xrc_ref, nbc_ref, batc_ref, valc_ref,
               xrr_ref, nbr_ref, batr_ref, valr_ref,
               cnt_ref, bconst_ref, x_ref, sel_ref, xout_ref):
    i = pl.program_id(0)
    bconst = bconst_ref[0, 0]
    s_r = xrc_ref[...] + nbc_ref[0] + nbc_ref[1] + bconst
    bat_r = batc_ref[...]
    val_r = valc_ref[...]
    idx_r = i * RBR + lax.broadcasted_iota(jnp.int32, (RBR, 1), 0)

    def body(cb, acc):
        dsl = pl.ds(cb * CBR, CBR)
        sc = (xrr_ref[0:1, dsl] + nbr_ref[0:1, dsl] + nbr_ref[1:2, dsl]
              + bconst)
        bc = batr_ref[0:1, dsl]
        vc = valr_ref[0:1, dsl]
        ic = cb * CBR + lax.broadcasted_iota(jnp.int32, (1, CBR), 1)
        beats = (sc > s_r) | ((sc == s_r) & (ic < idx_r))
        m = (bc == bat_r) & (vc > 0.0) & beats
        return acc + jnp.sum(m.astype(jnp.float32), axis=1, keepdims=True)

    rank = lax.fori_loop(
        cb_ref[i, 0], cb_ref[i, 1], body, jnp.zeros((RBR, 1), jnp.float32)
    )
    oh = (bat_r == lax.broadcasted_iota(jnp.int32, (1, G), 1)).astype(
        jnp.float32
    )
    kv = jnp.ceil(0.5 * cnt_ref[...])
    k_r = jnp.dot(oh, kv, preferred_element_type=jnp.float32)
    sel = (val_r > 0.0) & (rank < k_r)
    sel_ref[...] = sel.astype(jnp.float32)
    xout_ref[...] = x_ref[...] * jnp.where(sel, jnp.tanh(s_r), 0.0)


def _rank(xr_col, nbr_col, bat_col, val_col, xr_row, nbr_row, bat_row,
          val_row, counts, bconst, x, cb_bounds):
    return pl.pallas_call(
        _rank_body,
        grid=(NPAD // RBR,),
        in_specs=[
            pl.BlockSpec(memory_space=pltpu.SMEM),
            pl.BlockSpec((RBR, 1), lambda i: (i, 0)),
            pl.BlockSpec((2, RBR, 1), lambda i: (0, i, 0)),
            pl.BlockSpec((RBR, 1), lambda i: (i, 0)),
            pl.BlockSpec((RBR, 1), lambda i: (i, 0)),
            pl.BlockSpec((1, NPAD), lambda i: (0, 0)),
            pl.BlockSpec((2, NPAD), lambda i: (0, 0)),
            pl.BlockSpec((1, NPAD), lambda i: (0, 0)),
            pl.BlockSpec((1, NPAD), lambda i: (0, 0)),
            pl.BlockSpec((G, 1), lambda i: (0, 0)),
            pl.BlockSpec((1, 1), lambda i: (0, 0)),
            pl.BlockSpec((RBR, HID), lambda i: (i, 0)),
        ],
        out_specs=[
            pl.BlockSpec((RBR, 1), lambda i: (i, 0)),
            pl.BlockSpec((RBR, HID), lambda i: (i, 0)),
        ],
        out_shape=[
            jax.ShapeDtypeStruct((NPAD, 1), jnp.float32),
            jax.ShapeDtypeStruct((NPAD, HID), jnp.float32),
        ],
    )(cb_bounds, xr_col, nbr_col, bat_col, val_col, xr_row, nbr_row, bat_row,
      val_row, counts, bconst, x)


def _head_body(*refs):
    (s1, m1, c1, s2, m2, c2, s3, m3, c3, s4, m4, c4, s5, m5, c5, s6, m6, c6,
     w1, b1, w2, b2, w3, b3, o_ref) = refs

    def jk(s, m, c):
        mean = s[...] / jnp.maximum(c[...], 1.0)
        return jax.nn.gelu(jnp.concatenate([mean, m[...]], axis=1))

    cat0 = jk(s1, m1, c1) + jk(s2, m2, c2) + jk(s3, m3, c3)
    cat1 = jk(s4, m4, c4) + jk(s5, m5, c5) + jk(s6, m6, c6)
    z = jax.nn.gelu(jnp.concatenate([cat0, cat1], axis=1))
    z = jnp.maximum(
        jnp.dot(z, w1[...], preferred_element_type=jnp.float32) + b1[...], 0.0
    )
    z = jnp.maximum(
        jnp.dot(z, w2[...], preferred_element_type=jnp.float32) + b2[...], 0.0
    )
    o_ref[...] = (
        jnp.dot(z, w3[...], preferred_element_type=jnp.float32) + b3[...]
    )


def _head(pools, params):
    n_classes = params["l3_W"].shape[1]
    args = []
    for s, m, c in pools:
        args += [s, m, c]
    args += [params["l1_W"], params["l1_b"].reshape(1, -1),
             params["l2_W"], params["l2_b"].reshape(1, -1),
             params["l3_W"], params["l3_b"].reshape(1, -1)]
    return pl.pallas_call(
        _head_body,
        out_shape=jax.ShapeDtypeStruct((G, n_classes), jnp.float32),
    )(*args)


def _sc_agg_call(with_deg, v, src, dsteff, zeros2d, zeros1d, ones_ch):
    out_type = [jax.ShapeDtypeStruct((2, NPAD, HID), jnp.float32)]
    if with_deg:
        out_type.append(jax.ShapeDtypeStruct((2, NPAD), jnp.float32))

    def body(v_hbm, src_hbm, dst_hbm, z2_hbm, z1_hbm, ones_hbm, *rest):
        if with_deg:
            agg_out, deg_out = rest[0], rest[1]
            rest = rest[2:]
        else:
            agg_out = rest[0]
            deg_out = None
            rest = rest[1:]
        (si, di, rows, sit, dit, rowst, ones_v, onest_v, agg_s, deg_s,
         sem) = rest
        c = lax.axis_index("c")
        s = lax.axis_index("s")
        tb = s * RPT
        pltpu.sync_copy(z2_hbm.at[pl.ds(tb, RPT)], agg_s.at[pl.ds(tb, RPT)])
        if with_deg:
            pltpu.sync_copy(z1_hbm.at[pl.ds(tb, RPT)], deg_s.at[pl.ds(tb, RPT)])
            pltpu.sync_copy(ones_hbm, ones_v)
            pltpu.sync_copy(ones_hbm.at[pl.ds(0, TAIL)], onest_v)
        plsc.subcore_barrier()

        base = (c * 16 + s) * EPW

        def step(i, _):
            eb = base + i * CH
            pltpu.sync_copy(src_hbm.at[pl.ds(eb, CH)], si)
            pltpu.sync_copy(dst_hbm.at[pl.ds(eb, CH)], di)
            pltpu.async_copy(v_hbm.at[si], rows, sem).wait()
            pltpu.sync_copy(rows, agg_s.at[di], add=True)
            if with_deg:
                pltpu.sync_copy(ones_v, deg_s.at[di], add=True)
            return 0

        lax.fori_loop(0, NFULL, step, 0)
        eb = base + NFULL * CH
        pltpu.sync_copy(src_hbm.at[pl.ds(eb, TAIL)], sit)
        pltpu.sync_copy(dst_hbm.at[pl.ds(eb, TAIL)], dit)
        pltpu.async_copy(v_hbm.at[sit], rowst, sem).wait()
        pltpu.sync_copy(rowst, agg_s.at[dit], add=True)
        if with_deg:
            pltpu.sync_copy(onest_v, deg_s.at[dit], add=True)

        plsc.subcore_barrier()
        for j in range(RPT // CH):
            pltpu.sync_copy(
                agg_s.at[pl.ds(tb + j * CH, CH)],
                agg_out.at[c, pl.ds(tb + j * CH, CH)],
            )
        if with_deg:
            pltpu.sync_copy(
                deg_s.at[pl.ds(tb, RPT)], deg_out.at[c, pl.ds(tb, RPT)]
            )

    f = pl.kernel(
        body,
        out_type=out_type,
        mesh=_mesh(),
        scratch_types=[
            pltpu.VMEM((CH,), jnp.int32),
            pltpu.VMEM((CH,), jnp.int32),
            pltpu.VMEM((CH, HID), jnp.float32),
            pltpu.VMEM((TAIL,), jnp.int32),
            pltpu.VMEM((TAIL,), jnp.int32),
            pltpu.VMEM((TAIL, HID), jnp.float32),
            pltpu.VMEM((CH,), jnp.float32),
            pltpu.VMEM((TAIL,), jnp.float32),
            pltpu.VMEM_SHARED((NPAD, HID), jnp.float32),
            pltpu.VMEM_SHARED((NPAD,), jnp.float32),
            pltpu.SemaphoreType.DMA,
        ],
    )
    return f(v, src, dsteff, zeros2d, zeros1d, ones_ch)


def _sc_nbr_call(xn, src, dsteff, zeros1d):
    def body(xn_hbm, src_hbm, dst_hbm, z1_hbm, nbr_out, si, di, vals, sit,
             dit, valst, nbr_s, sem):
        c = lax.axis_index("c")
        s = lax.axis_index("s")
        tb = s * RPT
        pltpu.sync_copy(z1_hbm.at[pl.ds(tb, RPT)], nbr_s.at[pl.ds(tb, RPT)])
        plsc.subcore_barrier()

        base = (c * 16 + s) * EPW

        def step(i, _):
            eb = base + i * CH
            pltpu.sync_copy(src_hbm.at[pl.ds(eb, CH)], si)
            pltpu.sync_copy(dst_hbm.at[pl.ds(eb, CH)], di)
            pltpu.async_copy(xn_hbm.at[si], vals, sem).wait()
            pltpu.sync_copy(vals, nbr_s.at[di], add=True)
            return 0

        lax.fori_loop(0, NFULL, step, 0)
        eb = base + NFULL * CH
        pltpu.sync_copy(src_hbm.at[pl.ds(eb, TAIL)], sit)
        pltpu.sync_copy(dst_hbm.at[pl.ds(eb, TAIL)], dit)
        pltpu.async_copy(xn_hbm.at[sit], valst, sem).wait()
        pltpu.sync_copy(valst, nbr_s.at[dit], add=True)

        plsc.subcore_barrier()
        pltpu.sync_copy(nbr_s.at[pl.ds(tb, RPT)], nbr_out.at[c, pl.ds(tb, RPT)])

    f = pl.kernel(
        body,
        out_type=jax.ShapeDtypeStruct((2, NPAD), jnp.float32),
        mesh=_mesh(),
        scratch_types=[
            pltpu.VMEM((CH,), jnp.int32),
            pltpu.VMEM((CH,), jnp.int32),
            pltpu.VMEM((CH,), jnp.float32),
            pltpu.VMEM((TAIL,), jnp.int32),
            pltpu.VMEM((TAIL,), jnp.int32),
            pltpu.VMEM((TAIL,), jnp.float32),
            pltpu.VMEM_SHARED((NPAD,), jnp.float32),
            pltpu.SemaphoreType.DMA,
        ],
    )
    return f(xn, src, dsteff, zeros1d)


def _sc_dsteff_call(sel, src, dst):
    def body(sel_hbm, src_hbm, dst_hbm, out_hbm, si, di, sv, dv, ob, sit, dit,
             svt, dvt, obt, sem):
        c = lax.axis_index("c")
        s = lax.axis_index("s")
        base = (c * 16 + s) * EPW
        lane = lax.iota(jnp.int32, 16)

        def step(i, _):
            eb = base + i * CH
            pltpu.sync_copy(src_hbm.at[pl.ds(eb, CH)], si)
            pltpu.sync_copy(dst_hbm.at[pl.ds(eb, CH)], di)
            pltpu.async_copy(sel_hbm.at[si], sv, sem).wait()
            pltpu.async_copy(sel_hbm.at[di], dv, sem).wait()
            for j in range(CH // 16):
                sl = pl.ds(j * 16, 16)
                m = (sv[sl] > 0.0) & (dv[sl] > 0.0)
                sent = N + j * 16 + lane
                ob[sl] = jnp.where(m, di[sl], sent)
            pltpu.sync_copy(ob, out_hbm.at[pl.ds(eb, CH)])
            return 0

        lax.fori_loop(0, NFULL, step, 0)
        eb = base + NFULL * CH
        pltpu.sync_copy(src_hbm.at[pl.ds(eb, TAIL)], sit)
        pltpu.sync_copy(dst_hbm.at[pl.ds(eb, TAIL)], dit)
        pltpu.async_copy(sel_hbm.at[sit], svt, sem).wait()
        pltpu.async_copy(sel_hbm.at[dit], dvt, sem).wait()
        m = (svt[...] > 0.0) & (dvt[...] > 0.0)
        obt[...] = jnp.where(m, dit[...], N + lane)
        pltpu.sync_copy(obt, out_hbm.at[pl.ds(eb, TAIL)])

    f = pl.kernel(
        body,
        out_type=jax.ShapeDtypeStruct((E,), jnp.int32),
        mesh=_mesh(),
        scratch_types=[
            pltpu.VMEM((CH,), jnp.int32),
            pltpu.VMEM((CH,), jnp.int32),
            pltpu.VMEM((CH,), jnp.float32),
            pltpu.VMEM((CH,), jnp.float32),
            pltpu.VMEM((CH,), jnp.int32),
            pltpu.VMEM((TAIL,), jnp.int32),
            pltpu.VMEM((TAIL,), jnp.int32),
            pltpu.VMEM((TAIL,), jnp.float32),
            pltpu.VMEM((TAIL,), jnp.float32),
            pltpu.VMEM((TAIL,), jnp.int32),
            pltpu.SemaphoreType.DMA,
        ],
    )
    return f(sel, src, dst)


def kernel(x, edge_index, batch, params):
    src = edge_index[0].astype(jnp.int32)
    dst = edge_index[1].astype(jnp.int32)
    bat = batch.astype(jnp.int32)

    x_pad = jnp.pad(x, ((0, NPAD - N), (0, 0)))
    bat_pad = jnp.concatenate([bat, jnp.full((NPAD - N,), G, jnp.int32)])
    valid0 = jnp.concatenate(
        [jnp.ones((N,), jnp.float32), jnp.zeros((NPAD - N,), jnp.float32)]
    )

    gids = jnp.arange(G, dtype=jnp.int32)
    seg_lo = jnp.searchsorted(bat, gids, side="left").astype(jnp.int32)
    seg_hi = jnp.searchsorted(bat, gids, side="right").astype(jnp.int32)
    pblk = jnp.arange(NPAD // PRB, dtype=jnp.int32) * PRB
    pool_bounds = jnp.stack(
        [bat_pad[pblk], jnp.minimum(bat_pad[pblk + PRB - 1], G - 1)], axis=1
    )
    rblk = jnp.arange(NPAD // RBR, dtype=jnp.int32) * RBR
    glo_r = jnp.minimum(bat_pad[rblk], G - 1)
    ghi_r = jnp.minimum(bat_pad[rblk + RBR - 1], G - 1)
    cb_bounds = jnp.stack(
        [seg_lo[glo_r] // CBR, (seg_hi[ghi_r] + CBR - 1) // CBR], axis=1
    )

    zeros2d = jnp.zeros((NPAD, HID), jnp.float32)
    zeros1d = jnp.zeros((NPAD,), jnp.float32)
    ones_ch = jnp.ones((CH,), jnp.float32)

    bat_row = bat_pad.reshape(1, NPAD)
    bat_col = bat_pad.reshape(NPAD, 1)

    x_cur = _embed(x_pad, params["emb_W"], params["emb_b"])
    valid = valid0
    dsteff = dst
    pools = []
    counts_for_k = None
    sel_flat = None

    for h in range(2):
        hp = params["hier"][h]
        val_row = valid.reshape(1, NPAD)
        val_col = valid.reshape(NPAD, 1)
        pw = jnp.zeros((HID, 8), jnp.float32)
        pw = pw.at[:, 0].set(hp["pool"]["Wr"][:, 0])
        pw = pw.at[:, 1].set(hp["pool"]["Wn"][:, 0])

        deg2c = None
        hier_pools = []
        for li, lp in enumerate(hp["layers"]):
            u, v = _dense1(x_cur, lp)
            if li == 0:
                agg2, deg2 = _sc_agg_call(
                    True, v, src, dsteff, zeros2d, zeros1d, ones_ch
                )
                deg2c = deg2[:, :, None]
            else:
                (agg2,) = _sc_agg_call(
                    False, v, src, dsteff, zeros2d, zeros1d, ones_ch
                )
            last = li == len(hp["layers"]) - 1
            x_cur, xrn = _dense2(u, agg2, deg2c, x_cur, lp,
                                 pw if last else None)
            sums, mx, cnt = _pool(
                x_cur, bat_row, val_row, bat_col, val_col, pool_bounds
            )
            hier_pools.append((sums, mx, cnt))

        if counts_for_k is None:
            counts_for_k = hier_pools[0][2]

        xn_flat = xrn[:, 1]
        nbr2 = _sc_nbr_call(xn_flat, src, dsteff, zeros1d)

        xr_col = xrn[:, 0:1]
        xr_row = xrn[:, 0].reshape(1, NPAD)
        nbr_col = nbr2[:, :, None]
        bconst = hp["pool"]["b"].reshape(1, 1)
        sel_col, x_cur = _rank(
            xr_col, nbr_col, bat_col, val_col, xr_row, nbr2, bat_row,
            val_row, counts_for_k, bconst, x_cur, cb_bounds
        )
        sel_flat = sel_col[:, 0]

        sel_row = sel_flat.reshape(1, NPAD)
        sums, mx, cnt = _pool(
            x_cur, bat_row, sel_row, bat_col, sel_col, pool_bounds
        )
        hier_pools.append((sums, mx, cnt))
        pools.extend(hier_pools)

        counts_for_k = cnt
        valid = sel_flat
        if h == 0:
            dsteff = _sc_dsteff_call(sel_flat, src, dst)

    return _head(pools, params)
